# Initial kernel scaffold; baseline (speedup 1.0000x reference)
import jax
import jax.numpy as jnp
from jax.experimental import pallas as pl


def kernel(voltage, stimulus, neuron_type, edge_index, w, V_i_rest, tau_i):
    raise NotImplementedError("write your pallas kernel here")


# speedup vs baseline: 148.1706x; 148.1706x over previous
import functools

import jax
import jax.numpy as jnp
from jax import lax
from jax.experimental import pallas as pl
from jax.experimental.pallas import tpu as pltpu
from jax.experimental.pallas import tpu_sc as plsc

N = 100000
E = 6400000
NC = 2
NS = 16
NW = NC * NS
EPW = E // NW
CHUNK = 2000
NSTEPS = EPW // CHUNK
ACC_P = 100352
SLICE = ACC_P // NS
LANES = 16

_mesh = plsc.VectorSubcoreMesh(core_axis_name="c", subcore_axis_name="s")


@functools.partial(
    pl.kernel,
    out_type=jax.ShapeDtypeStruct((NC, ACC_P), jnp.float32),
    mesh=_mesh,
    compiler_params=pltpu.CompilerParams(needs_layout_passes=False),
    scratch_types=[
        pltpu.VMEM((N,), jnp.float32),
        pltpu.VMEM((CHUNK,), jnp.int32),
        pltpu.VMEM((CHUNK,), jnp.int32),
        pltpu.VMEM((CHUNK,), jnp.float32),
        pltpu.VMEM((CHUNK,), jnp.float32),
        pltpu.VMEM((SLICE,), jnp.float32),
        pltpu.VMEM_SHARED((ACC_P,), jnp.float32),
    ],
)
def _scatter_add_sc(edge_hbm, w_hbm, v_hbm, out_hbm,
                    v_v, src_v, dst_v, w_v, msg_v, z_v, acc_sh):
    c = lax.axis_index("c")
    s = lax.axis_index("s")
    wid = s * NC + c

    pltpu.sync_copy(v_hbm, v_v)

    def _zero(i, carry):
        z_v[pl.ds(i * LANES, LANES)] = jnp.zeros((LANES,), jnp.float32)
        return carry
    lax.fori_loop(0, SLICE // LANES, _zero, 0, unroll=8)
    pltpu.sync_copy(z_v, acc_sh.at[pl.ds(s * SLICE, SLICE)])
    plsc.subcore_barrier()

    base0 = wid * EPW

    def _step(k, carry):
        base = base0 + k * CHUNK
        pltpu.sync_copy(edge_hbm.at[pl.ds(base, CHUNK)], src_v)
        pltpu.sync_copy(edge_hbm.at[pl.ds(E + base, CHUNK)], dst_v)
        pltpu.sync_copy(w_hbm.at[pl.ds(base, CHUNK)], w_v)

        def _gather(i, inner):
            sl = pl.ds(i * LANES, LANES)
            vs = plsc.load_gather(v_v, [src_v[sl]])
            msg_v[sl] = w_v[sl] * jnp.maximum(vs, 0.0)
            return inner
        lax.fori_loop(0, CHUNK // LANES, _gather, 0, unroll=4)

        pltpu.sync_copy(msg_v, acc_sh.at[dst_v], add=True)
        return carry

    lax.fori_loop(0, NSTEPS, _step, 0)

    plsc.subcore_barrier()
    sl = pl.ds(s * SLICE, SLICE)
    pltpu.sync_copy(acc_sh.at[sl], out_hbm.at[c, sl])


_ROWS = ACC_P // 128


def _combine_body(p_ref, v_ref, e_ref, r_ref, t_ref, o_ref):
    msg = p_ref[0] + p_ref[1]
    o_ref[...] = (msg - v_ref[...] + e_ref[...] + r_ref[...]) / t_ref[...]


def kernel(voltage, stimulus, neuron_type, edge_index, w, V_i_rest, tau_i):
    del neuron_type
    partial = _scatter_add_sc(edge_index.reshape(-1), w, voltage)

    pad = ACC_P - N
    vp = jnp.pad(voltage, (0, pad)).reshape(_ROWS, 128)
    ep = jnp.pad(stimulus, (0, pad)).reshape(_ROWS, 128)
    rp = jnp.pad(V_i_rest, (0, pad)).reshape(_ROWS, 128)
    tp = jnp.pad(tau_i, (0, pad), constant_values=1.0).reshape(_ROWS, 128)
    pr = partial.reshape(NC, _ROWS, 128)

    dv = pl.pallas_call(
        _combine_body,
        out_shape=jax.ShapeDtypeStruct((_ROWS, 128), jnp.float32),
    )(pr, vp, ep, rp, tp)
    return dv.reshape(-1)[:N, None]


# speedup vs baseline: 323.6896x; 2.1846x over previous
import functools

import jax
import jax.numpy as jnp
from jax import lax
from jax.experimental import pallas as pl
from jax.experimental.pallas import tpu as pltpu
from jax.experimental.pallas import tpu_sc as plsc

N = 100000
E = 6400000
NC = 2
NS = 16
NW = NC * NS
EPW = E // NW
CHUNK = 2000
NSTEPS = EPW // CHUNK
ACC_P = 100352
SLICE = ACC_P // NS
QUART = SLICE // 4
LANES = 16

_mesh = plsc.VectorSubcoreMesh(core_axis_name="c", subcore_axis_name="s")


@functools.partial(
    pl.kernel,
    out_type=jax.ShapeDtypeStruct((NC, ACC_P), jnp.float32),
    mesh=_mesh,
    compiler_params=pltpu.CompilerParams(needs_layout_passes=False),
    scratch_types=[
        pltpu.VMEM((N,), jnp.float32),
        pltpu.VMEM((CHUNK,), jnp.int32),
        pltpu.VMEM((CHUNK,), jnp.int32),
        pltpu.VMEM((CHUNK,), jnp.float32),
        pltpu.VMEM((CHUNK,), jnp.float32),
        pltpu.VMEM((CHUNK,), jnp.int32),
        pltpu.VMEM((CHUNK,), jnp.int32),
        pltpu.VMEM((CHUNK,), jnp.int32),
        pltpu.VMEM((CHUNK,), jnp.int32),
        pltpu.VMEM((CHUNK,), jnp.float32),
        pltpu.VMEM((CHUNK,), jnp.float32),
        pltpu.VMEM((CHUNK,), jnp.float32),
        pltpu.VMEM((CHUNK,), jnp.float32),
        pltpu.VMEM_SHARED((ACC_P,), jnp.float32),
        pltpu.SemaphoreType.DMA,
        pltpu.SemaphoreType.DMA,
        pltpu.SemaphoreType.DMA,
        pltpu.SemaphoreType.DMA,
        pltpu.SemaphoreType.DMA,
        pltpu.SemaphoreType.DMA,
        pltpu.SemaphoreType.DMA,
    ],
)
def _scatter_add_sc(edge_hbm, w_hbm, v_hbm, out_hbm,
                    v_v, src0, src1, wv0, wv1,
                    dst0, dst1, dst2, dst3, msg0, msg1, msg2, msg3,
                    acc_sh, sem_v, sem_i0, sem_i1,
                    sem_s0, sem_s1, sem_s2, sem_s3):
    srcs = (src0, src1)
    wvs = (wv0, wv1)
    dsts = (dst0, dst1, dst2, dst3)
    msgs = (msg0, msg1, msg2, msg3)
    sems_i = (sem_i0, sem_i1)
    sems_s = (sem_s0, sem_s1, sem_s2, sem_s3)

    c = lax.axis_index("c")
    s = lax.axis_index("s")
    wid = s * NC + c
    base0 = wid * EPW

    v_desc = pltpu.async_copy(v_hbm, v_v, sem_v)

    def issue_in(step, j2, j4):
        base = base0 + step * CHUNK
        pltpu.async_copy(edge_hbm.at[pl.ds(base, CHUNK)], srcs[j2], sems_i[j2])
        pltpu.async_copy(edge_hbm.at[pl.ds(E + base, CHUNK)], dsts[j4],
                         sems_i[j2])
        pltpu.async_copy(w_hbm.at[pl.ds(base, CHUNK)], wvs[j2], sems_i[j2])

    def wait_in(j2, j4):
        pltpu.make_async_copy(edge_hbm.at[pl.ds(0, CHUNK)], srcs[j2],
                              sems_i[j2]).wait()
        pltpu.make_async_copy(edge_hbm.at[pl.ds(0, CHUNK)], dsts[j4],
                              sems_i[j2]).wait()
        pltpu.make_async_copy(w_hbm.at[pl.ds(0, CHUNK)], wvs[j2],
                              sems_i[j2]).wait()

    def wait_scatter(j4):
        pltpu.make_async_copy(msgs[j4], acc_sh.at[dsts[j4]], sems_s[j4]).wait()

    issue_in(0, 0, 0)
    issue_in(1, 1, 1)

    def _zero(i, carry):
        msg0[pl.ds(i * LANES, LANES)] = jnp.zeros((LANES,), jnp.float32)
        return carry
    lax.fori_loop(0, QUART // LANES, _zero, 0, unroll=8)
    for q in range(4):
        pltpu.sync_copy(msg0.at[pl.ds(0, QUART)],
                        acc_sh.at[pl.ds(s * SLICE + q * QUART, QUART)])
    plsc.subcore_barrier()
    v_desc.wait()

    def body(kk, carry):
        for j in range(4):
            step = kk * 4 + j
            j2 = j % 2
            wait_in(j2, j)

            def _gather(i, inner):
                sl = pl.ds(i * LANES, LANES)
                vs = plsc.load_gather(v_v, [srcs[j2][sl]])
                msgs[j][sl] = wvs[j2][sl] * jnp.maximum(vs, 0.0)
                return inner
            lax.fori_loop(0, CHUNK // LANES, _gather, 0, unroll=4)

            pltpu.async_copy(msgs[j], acc_sh.at[dsts[j]], sems_s[j], add=True)

            @pl.when(step >= 2)
            def _():
                wait_scatter((j + 2) % 4)

            @pl.when(step + 2 < NSTEPS)
            def _():
                issue_in(step + 2, j2, (j + 2) % 4)
        return carry

    lax.fori_loop(0, NSTEPS // 4, body, 0)

    wait_scatter(2)
    wait_scatter(3)
    plsc.subcore_barrier()
    sl_ = pl.ds(s * SLICE, SLICE)
    pltpu.sync_copy(acc_sh.at[sl_], out_hbm.at[c, sl_])


_ROWS = ACC_P // 128


def _combine_body(p_ref, v_ref, e_ref, r_ref, t_ref, o_ref):
    msg = p_ref[0] + p_ref[1]
    o_ref[...] = (msg - v_ref[...] + e_ref[...] + r_ref[...]) / t_ref[...]


def kernel(voltage, stimulus, neuron_type, edge_index, w, V_i_rest, tau_i):
    del neuron_type
    partial = _scatter_add_sc(edge_index.reshape(-1), w, voltage)

    pad = ACC_P - N
    vp = jnp.pad(voltage, (0, pad)).reshape(_ROWS, 128)
    ep = jnp.pad(stimulus, (0, pad)).reshape(_ROWS, 128)
    rp = jnp.pad(V_i_rest, (0, pad)).reshape(_ROWS, 128)
    tp = jnp.pad(tau_i, (0, pad), constant_values=1.0).reshape(_ROWS, 128)
    pr = partial.reshape(NC, _ROWS, 128)

    dv = pl.pallas_call(
        _combine_body,
        out_shape=jax.ShapeDtypeStruct((_ROWS, 128), jnp.float32),
    )(pr, vp, ep, rp, tp)
    return dv.reshape(-1)[:N, None]
